# Initial kernel scaffold; baseline (speedup 1.0000x reference)
import jax
import jax.numpy as jnp
from jax.experimental import pallas as pl


def kernel(X, batch, num_graphs):
    raise NotImplementedError("write your pallas kernel here")


# speedup vs baseline: 3.7421x; 3.7421x over previous
import functools

import jax
import jax.numpy as jnp
from jax import lax
from jax.experimental import pallas as pl
from jax.experimental.pallas import tpu as pltpu
from jax.experimental.pallas import tpu_sc as plsc

N_ROWS = 320000
D = 128
NUM_GRAPHS = 512
LANES = 16
N_SUBCORES = 16
CHUNK = N_ROWS // N_SUBCORES
ACC = NUM_GRAPHS * LANES
G_PER_TILE = NUM_GRAPHS // N_SUBCORES

_R = 64


def _energy_body(x_ref, o_ref):
    x = x_ref[...]
    o_ref[...] = 0.5 * jnp.sum(x * x, axis=-1)


_energy = pl.pallas_call(
    _energy_body,
    grid=(pl.cdiv(2500, _R),),
    in_specs=[pl.BlockSpec((_R, D, D), lambda i: (i, 0, 0))],
    out_specs=pl.BlockSpec((_R, D), lambda i: (i, 0)),
    out_shape=jax.ShapeDtypeStruct((2500, D), jnp.float32),
    compiler_params=pltpu.CompilerParams(
        dimension_semantics=("arbitrary",),
    ),
)

_mesh = plsc.VectorSubcoreMesh(
    core_axis_name="c", subcore_axis_name="s", num_cores=1
)


@functools.partial(
    pl.kernel,
    mesh=_mesh,
    out_type=jax.ShapeDtypeStruct((NUM_GRAPHS,), jnp.float32),
    scratch_types=[
        pltpu.VMEM((CHUNK,), jnp.float32),
        pltpu.VMEM((CHUNK,), jnp.int32),
        pltpu.VMEM((ACC,), jnp.float32),
        pltpu.VMEM((G_PER_TILE * LANES,), jnp.float32),
        pltpu.VMEM((G_PER_TILE * LANES,), jnp.float32),
        pltpu.VMEM((G_PER_TILE,), jnp.float32),
        pltpu.VMEM_SHARED((N_SUBCORES * ACC,), jnp.float32),
    ],
    compiler_params=pltpu.CompilerParams(needs_layout_passes=False),
)
def _scatter(e_hbm, b_hbm, out_hbm, e_v, b_v, acc_v, sum_v, stage_v,
             res_v, shared):
    sid = lax.axis_index("s")
    base = sid * CHUNK

    pltpu.sync_copy(e_hbm.at[pl.ds(base, CHUNK)], e_v)
    pltpu.sync_copy(b_hbm.at[pl.ds(base, CHUNK)], b_v)

    zeros16 = jnp.zeros((LANES,), jnp.float32)

    def _zero(i, _):
        acc_v[pl.ds(i * LANES, LANES)] = zeros16
        return 0

    lax.fori_loop(0, ACC // LANES, _zero, 0)

    lane = lax.iota(jnp.int32, LANES)

    def _accum(i, _):
        idx = b_v[pl.ds(i * LANES, LANES)]
        ev = e_v[pl.ds(i * LANES, LANES)]
        plsc.addupdate_scatter(acc_v, [idx * LANES + lane], ev)
        return 0

    lax.fori_loop(0, CHUNK // LANES, _accum, 0)

    pltpu.sync_copy(acc_v, shared.at[pl.ds(sid * ACC, ACC)])
    plsc.subcore_barrier()

    goff = sid * G_PER_TILE * LANES

    def _zero2(i, _):
        sum_v[pl.ds(i * LANES, LANES)] = zeros16
        return 0

    lax.fori_loop(0, G_PER_TILE, _zero2, 0)

    def _gather_tile(t, _):
        pltpu.sync_copy(
            shared.at[pl.ds(t * ACC + goff, G_PER_TILE * LANES)], stage_v
        )
        for c in range(G_PER_TILE):
            s = pl.ds(c * LANES, LANES)
            sum_v[s] = sum_v[s] + stage_v[s]
        return 0

    lax.fori_loop(0, N_SUBCORES, _gather_tile, 0)

    for c in range(G_PER_TILE // LANES):
        addr = c * LANES * LANES + lane * LANES
        tot = plsc.load_gather(sum_v, [addr])
        for l in range(1, LANES):
            tot = tot + plsc.load_gather(sum_v, [addr + l])
        res_v[pl.ds(c * LANES, LANES)] = tot

    pltpu.sync_copy(res_v, out_hbm.at[pl.ds(sid * G_PER_TILE, G_PER_TILE)])


def kernel(X, batch, num_graphs):
    del num_graphs
    e = _energy(X.reshape(2500, D, D)).reshape(N_ROWS)
    return _scatter(e, batch.astype(jnp.int32))


# speedup vs baseline: 6.2540x; 1.6713x over previous
import functools

import jax
import jax.numpy as jnp
from jax import lax
from jax.experimental import pallas as pl
from jax.experimental.pallas import tpu as pltpu
from jax.experimental.pallas import tpu_sc as plsc

N_ROWS = 320000
D = 128
NUM_GRAPHS = 512
LANES = 16
N_SUBCORES = 16
CHUNK = N_ROWS // N_SUBCORES
ACC = NUM_GRAPHS * LANES
G_PER_TILE = NUM_GRAPHS // N_SUBCORES

_R = 64


def _energy_body(x_ref, o_ref):
    x = x_ref[...]
    o_ref[...] = 0.5 * jnp.sum(x * x, axis=-1)


_energy = pl.pallas_call(
    _energy_body,
    grid=(pl.cdiv(2500, _R),),
    in_specs=[pl.BlockSpec((_R, D, D), lambda i: (i, 0, 0))],
    out_specs=pl.BlockSpec((_R, D), lambda i: (i, 0)),
    out_shape=jax.ShapeDtypeStruct((2500, D), jnp.float32),
    compiler_params=pltpu.CompilerParams(
        dimension_semantics=("arbitrary",),
    ),
)

_mesh = plsc.VectorSubcoreMesh(
    core_axis_name="c", subcore_axis_name="s", num_cores=1
)


@functools.partial(
    pl.kernel,
    mesh=_mesh,
    out_type=jax.ShapeDtypeStruct((NUM_GRAPHS,), jnp.float32),
    scratch_types=[
        pltpu.VMEM((CHUNK,), jnp.float32),
        pltpu.VMEM((CHUNK,), jnp.int32),
        pltpu.VMEM((ACC,), jnp.float32),
        pltpu.VMEM((G_PER_TILE * LANES,), jnp.float32),
        pltpu.VMEM((G_PER_TILE * LANES,), jnp.float32),
        pltpu.VMEM((G_PER_TILE,), jnp.float32),
        pltpu.VMEM_SHARED((N_SUBCORES * ACC,), jnp.float32),
    ],
    compiler_params=pltpu.CompilerParams(needs_layout_passes=False),
)
def _scatter(e_hbm, b_hbm, out_hbm, e_v, b_v, acc_v, sum_v, stage_v,
             res_v, shared):
    sid = lax.axis_index("s")
    base = sid * CHUNK

    pltpu.sync_copy(e_hbm.at[pl.ds(base, CHUNK)], e_v)
    pltpu.sync_copy(b_hbm.at[pl.ds(base, CHUNK)], b_v)

    zeros16 = jnp.zeros((LANES,), jnp.float32)

    @plsc.parallel_loop(0, ACC // LANES, unroll=8)
    def _zero(i):
        acc_v[pl.ds(i * LANES, LANES)] = zeros16

    lane = lax.iota(jnp.int32, LANES)

    @plsc.parallel_loop(0, CHUNK // LANES, unroll=8)
    def _accum(i):
        s = pl.ds(i * LANES, LANES)
        idx = b_v[s]
        ev = e_v[s]
        plsc.addupdate_scatter(acc_v, [idx * LANES + lane], ev)

    pltpu.sync_copy(acc_v, shared.at[pl.ds(sid * ACC, ACC)])
    plsc.subcore_barrier()

    goff = sid * G_PER_TILE * LANES

    def _zero2(i, _):
        sum_v[pl.ds(i * LANES, LANES)] = zeros16
        return 0

    lax.fori_loop(0, G_PER_TILE, _zero2, 0)

    def _gather_tile(t, _):
        pltpu.sync_copy(
            shared.at[pl.ds(t * ACC + goff, G_PER_TILE * LANES)], stage_v
        )
        for c in range(G_PER_TILE):
            s = pl.ds(c * LANES, LANES)
            sum_v[s] = sum_v[s] + stage_v[s]
        return 0

    lax.fori_loop(0, N_SUBCORES, _gather_tile, 0)

    for c in range(G_PER_TILE // LANES):
        addr = c * LANES * LANES + lane * LANES
        tot = plsc.load_gather(sum_v, [addr])
        for l in range(1, LANES):
            tot = tot + plsc.load_gather(sum_v, [addr + l])
        res_v[pl.ds(c * LANES, LANES)] = tot

    pltpu.sync_copy(res_v, out_hbm.at[pl.ds(sid * G_PER_TILE, G_PER_TILE)])


def kernel(X, batch, num_graphs):
    del num_graphs
    e = _energy(X.reshape(2500, D, D)).reshape(N_ROWS)
    return _scatter(e, batch.astype(jnp.int32))
